# Initial kernel scaffold; baseline (speedup 1.0000x reference)
import jax
import jax.numpy as jnp
from jax.experimental import pallas as pl


def kernel(x, edge_index, edge_weight, W1, b1, W2, b2, W3, b3, W4, b4, W5, b5):
    raise NotImplementedError("write your pallas kernel here")


# speedup vs baseline: 7.5594x; 7.5594x over previous
import functools

import jax
import jax.numpy as jnp
import numpy as np
from jax import lax
from jax.experimental import pallas as pl
from jax.experimental.pallas import tpu as pltpu
from jax.experimental.pallas import tpu_sc as plsc

N = 10000
E = 160000
K = 20

NC = 2
NS = 16
NPT = N // NS
EPT = E // NS
EB = 80
NBLK = EPT // EB

ZR = 25
ND = 10240
NDT = ND // NS

F32 = jnp.float32
I32 = jnp.int32


def _i32(v):
    return jnp.int32(v)


_z = np.int32(0)


def _fori(lo, hi, body):
    lax.fori_loop(jnp.int32(lo), jnp.int32(hi), lambda i, _: body(i) or 0, 0)

_MESH = plsc.VectorSubcoreMesh(core_axis_name="c", subcore_axis_name="s")
_SC_PARAMS = pltpu.CompilerParams(needs_layout_passes=False,
                                  use_tc_tiling_on_sc=False)


def _zero_rows(ref, nrows, ncols):
    zv = jnp.zeros((16,), F32)

    def row(r):
        for j in range(ncols // 16):
            ref[r, pl.ds(j * 16, 16)] = zv

    _fori(0, nrows, row)


def _rsqrt_or_zero(v):
    y = plsc.bitcast(jnp.int32(0x5F3759DF) - (plsc.bitcast(v, I32) >> 1), F32)
    for _ in range(3):
        y = y * (1.5 - 0.5 * v * y * y)
    return jnp.where(v > 0.0, y, jnp.zeros((16,), F32))


def _norm_call(row3, col3, ew3):
    def body(row_h, col_h, ew_h, norm_h, deg_sp, row_v, col_v, ew_v, nrm_v,
             deg_v, zb):
        c = lax.axis_index("c")
        s = lax.axis_index("s")

        pltpu.sync_copy(row_h.at[s], row_v)
        pltpu.sync_copy(col_h.at[s], col_v)
        pltpu.sync_copy(ew_h.at[s], ew_v)

        zv = jnp.zeros((16,), F32)
        for j in range(NDT // 16):
            zb[pl.ds(j * 16, 16)] = zv
        pltpu.sync_copy(zb, deg_sp.at[pl.ds(s * _i32(NDT), NDT)])
        plsc.subcore_barrier()

        def blk_deg(blk):
            pltpu.sync_copy(ew_v.at[blk], deg_sp.at[col_v.at[blk]], add=True)

        _fori(0, NBLK, blk_deg)
        plsc.subcore_barrier()

        pltpu.sync_copy(deg_sp, deg_v)

        def chunk(i):
            v = deg_v[pl.ds(i * 16, 16)]
            deg_v[pl.ds(i * 16, 16)] = _rsqrt_or_zero(v)

        _fori(0, ND // 16, chunk)

        def blk_norm(blk):
            for j in range(EB // 16):
                r = row_v[blk, pl.ds(j * 16, 16)]
                cc = col_v[blk, pl.ds(j * 16, 16)]
                dr = plsc.load_gather(deg_v, [r])
                dc = plsc.load_gather(deg_v, [cc])
                nrm_v[blk, pl.ds(j * 16, 16)] = (
                    dr * dc * ew_v[blk, pl.ds(j * 16, 16)])

        _fori(0, NBLK, blk_norm)

        @pl.when(c == 0)
        def _():
            pltpu.sync_copy(nrm_v, norm_h.at[s])

    f = pl.kernel(
        body,
        out_type=jax.ShapeDtypeStruct((NS, NBLK, EB), F32),
        mesh=_MESH,
        compiler_params=_SC_PARAMS,
        scratch_types=[
            pltpu.VMEM_SHARED((ND,), F32),
            pltpu.VMEM((NBLK, EB), I32),
            pltpu.VMEM((NBLK, EB), I32),
            pltpu.VMEM((NBLK, EB), F32),
            pltpu.VMEM((NBLK, EB), F32),
            pltpu.VMEM((ND,), F32),
            pltpu.VMEM((NDT,), F32),
        ],
    )
    return f(row3, col3, ew3)


def _make_hop_call(fcw):
    def body(h_h, row_h, col_h, nrm_h, t_h, acc_sp, row_v, col_v, nrm_v,
             gbuf, zbuf):
        c = lax.axis_index("c")
        s = lax.axis_index("s")

        pltpu.sync_copy(row_h.at[s], row_v)
        pltpu.sync_copy(col_h.at[s], col_v)
        pltpu.sync_copy(nrm_h.at[s], nrm_v)

        _zero_rows(zbuf, ZR, fcw)

        def zero_acc():
            def zi(i):
                pltpu.sync_copy(
                    zbuf, acc_sp.at[pl.ds(s * _i32(NPT) + i * ZR, ZR)])
            _fori(0, NPT // ZR, zi)

        zero_acc()
        plsc.subcore_barrier()

        def do_hop(src, k_out):
            def blk_body(blk):
                pltpu.sync_copy(src.at[row_v.at[blk]], gbuf)

                def e_body(e):
                    nv = plsc.load_gather(
                        nrm_v, [jnp.full((16,), blk, I32),
                                jnp.full((16,), e, I32)])
                    for j in range(fcw // 16):
                        sl = pl.ds(j * 16, 16)
                        gbuf[e, sl] = gbuf[e, sl] * nv

                _fori(0, EB, e_body)
                pltpu.sync_copy(gbuf, acc_sp.at[col_v.at[blk]], add=True)

            _fori(0, NBLK, blk_body)
            plsc.subcore_barrier()
            sl = pl.ds(s * _i32(NPT), NPT)
            pltpu.sync_copy(acc_sp.at[sl], t_h.at[k_out, c, sl])
            zero_acc()
            plsc.subcore_barrier()

        do_hop(h_h.at[c], _i32(0))

        def k_body(k):
            do_hop(t_h.at[k - 1, c], k)

        _fori(1, K, k_body)

    f = pl.kernel(
        body,
        out_type=jax.ShapeDtypeStruct((K, NC, N, fcw), F32),
        mesh=_MESH,
        compiler_params=_SC_PARAMS,
        scratch_types=[
            pltpu.VMEM_SHARED((N, fcw), F32),
            pltpu.VMEM((NBLK, EB), I32),
            pltpu.VMEM((NBLK, EB), I32),
            pltpu.VMEM((NBLK, EB), F32),
            pltpu.VMEM((EB, fcw), F32),
            pltpu.VMEM((ZR, fcw), F32),
        ],
    )
    return f


_HOP_CALLS = {}


def _hop(fcw, h0, row3, col3, nrm3):
    if fcw not in _HOP_CALLS:
        _HOP_CALLS[fcw] = _make_hop_call(fcw)
    return _HOP_CALLS[fcw](h0, row3, col3, nrm3)


def _tc_layer(h, T, Wp, bp, fcw_in, fcw_out, act):
    FO = 2 * fcw_out
    BN = 400
    G = N // BN

    def body(h_ref, t_ref, w_ref, b_ref, o_ref):
        acc = jnp.dot(h_ref[0], w_ref[0, 0], preferred_element_type=F32)
        acc = acc + jnp.dot(h_ref[1], w_ref[0, 1], preferred_element_type=F32)

        def kb(k, a):
            a = a + jnp.dot(t_ref[k, 0], w_ref[k + 1, 0],
                            preferred_element_type=F32)
            a = a + jnp.dot(t_ref[k, 1], w_ref[k + 1, 1],
                            preferred_element_type=F32)
            return a

        acc = lax.fori_loop(jnp.int32(0), jnp.int32(K), kb, acc)
        acc = acc + b_ref[0]
        if act == "relu":
            acc = jnp.maximum(acc, 0.0)
        else:
            acc = jax.nn.sigmoid(acc)
        o_ref[0] = acc[:, :fcw_out]
        o_ref[1] = acc[:, fcw_out:]

    return pl.pallas_call(
        body,
        grid=(G,),
        in_specs=[
            pl.BlockSpec((NC, BN, fcw_in), lambda i: (_z, i, _z)),
            pl.BlockSpec((K, NC, BN, fcw_in), lambda i: (_z, _z, i, _z)),
            pl.BlockSpec((K + 1, NC, fcw_in, FO),
                         lambda i: (_z, _z, _z, _z)),
            pl.BlockSpec((1, FO), lambda i: (_z, _z)),
        ],
        out_specs=pl.BlockSpec((NC, BN, fcw_out), lambda i: (_z, i, _z)),
        out_shape=jax.ShapeDtypeStruct((NC, N, fcw_out), F32),
    )(h, T, Wp, bp)


FCW = {1: 16, 60: 32, 100: 64, 200: 112, 80: 48}


def _pack_w(W, b, fcw_in, fcw_out):
    kk, fi, fo = W.shape
    FO = 2 * fcw_out
    Wp = jnp.zeros((kk, 2 * fcw_in, FO), F32)
    Wp = Wp.at[:, :fi, :fo].set(W.astype(F32))
    Wp = Wp.reshape(kk, 2, fcw_in, FO)
    bp = jnp.zeros((1, FO), F32).at[0, :fo].set(b.astype(F32))
    return Wp, bp


def kernel(x, edge_index, edge_weight, W1, b1, W2, b2, W3, b3, W4, b4,
           W5, b5):
    row3 = edge_index[0].astype(I32).reshape(NS, NBLK, EB)
    col3 = edge_index[1].astype(I32).reshape(NS, NBLK, EB)
    ew3 = edge_weight.astype(F32).reshape(NS, NBLK, EB)

    nrm3 = _norm_call(row3, col3, ew3)

    h = jnp.zeros((NC, N, FCW[1]), F32).at[0, :, 0].set(x.astype(F32))

    layers = [(W1, b1, "relu"), (W2, b2, "relu"), (W3, b3, "relu"),
              (W4, b4, "relu"), (W5, b5, "sigmoid")]
    dims = [1, 60, 100, 200, 80, 1]
    for i, (W, b, act) in enumerate(layers):
        fcw_in = FCW[dims[i]]
        fcw_out = FCW[dims[i + 1]] if i < 4 else 8
        T = _hop(fcw_in, h, row3, col3, nrm3)
        Wp, bp = _pack_w(W, b, fcw_in, fcw_out)
        h = _tc_layer(h, T, Wp, bp, fcw_in, fcw_out, act)

    return h[0, :, 0]
